# Initial kernel scaffold; baseline (speedup 1.0000x reference)
import jax
import jax.numpy as jnp
from jax.experimental import pallas as pl


def kernel(feat, feat_a, adj, graph_neigh, W1, b1, W2, b2, Wb, bb):
    raise NotImplementedError("write your pallas kernel here")


# speedup vs baseline: 1.4768x; 1.4768x over previous
import functools

import jax
import jax.numpy as jnp
from jax.experimental import pallas as pl

N = 4096
BM = 256


def _prep_kernel(adj_ref, feat_ref, feata_ref, w1_ref, dinv_ref, y1s_ref):
    i = pl.program_id(0)
    adj = adj_ref[...]
    rows = jax.lax.broadcasted_iota(jnp.int32, adj.shape, 0) + i * BM
    cols = jax.lax.broadcasted_iota(jnp.int32, adj.shape, 1)
    b = jnp.where(cols == rows, 1.0, (adj != 0).astype(jnp.float32))
    deg = jnp.sum(b, axis=1, keepdims=True)
    dinv = jax.lax.rsqrt(deg)
    dinv_ref[...] = dinv
    xw = jnp.dot(feat_ref[...], w1_ref[...], preferred_element_type=jnp.float32)
    xwa = jnp.dot(feata_ref[...], w1_ref[...], preferred_element_type=jnp.float32)
    y1s_ref[...] = jnp.concatenate([xw, xwa], axis=1) * dinv


def _layer1_kernel(adj_ref, y1s_ref, dinv_ref, b1_ref, w2_ref, y2s_ref):
    i = pl.program_id(0)
    adj = adj_ref[...]
    rows = jax.lax.broadcasted_iota(jnp.int32, adj.shape, 0) + i * BM
    cols = jax.lax.broadcasted_iota(jnp.int32, adj.shape, 1)
    b = jnp.where(cols == rows, 1.0, (adj != 0).astype(jnp.float32))
    h = jnp.dot(b, y1s_ref[...], preferred_element_type=jnp.float32)
    dinv = dinv_ref[...]
    z = jax.nn.relu(h * dinv + b1_ref[...])
    hdim = w2_ref.shape[0]
    y2 = jnp.dot(z[:, :hdim], w2_ref[...], preferred_element_type=jnp.float32)
    y2a = jnp.dot(z[:, hdim:], w2_ref[...], preferred_element_type=jnp.float32)
    y2s_ref[...] = jnp.concatenate([y2, y2a], axis=1) * dinv


def _layer2_kernel(adj_ref, y2s_ref, dinv_ref, b2_ref, emb_ref):
    i = pl.program_id(0)
    adj = adj_ref[...]
    rows = jax.lax.broadcasted_iota(jnp.int32, adj.shape, 0) + i * BM
    cols = jax.lax.broadcasted_iota(jnp.int32, adj.shape, 1)
    b = jnp.where(cols == rows, 1.0, (adj != 0).astype(jnp.float32))
    h = jnp.dot(b, y2s_ref[...], preferred_element_type=jnp.float32)
    emb_ref[...] = h * dinv_ref[...] + b2_ref[...]


def _readout_kernel(g_ref, embcat_ref, embblk_ref, wb_ref, bb_ref,
                    ret_ref, reta_ref):
    gm = g_ref[...]
    vsum = jnp.dot(gm, embcat_ref[...], preferred_element_type=jnp.float32)
    rs = jnp.sum(gm, axis=1, keepdims=True)
    ge = vsum / rs
    d = ge.shape[1] // 2
    ge1, ge2 = ge[:, :d], ge[:, d:]
    n1 = jnp.sqrt(jnp.sum(ge1 * ge1, axis=1, keepdims=True))
    n2 = jnp.sqrt(jnp.sum(ge2 * ge2, axis=1, keepdims=True))
    g = jax.nn.sigmoid(ge1 / jnp.maximum(n1, 1e-12))
    ga = jax.nn.sigmoid(ge2 / jnp.maximum(n2, 1e-12))
    embblk = embblk_ref[...]
    emb, emba = embblk[:, :d], embblk[:, d:]
    t = jnp.dot(emb, wb_ref[...], preferred_element_type=jnp.float32)
    ta = jnp.dot(emba, wb_ref[...], preferred_element_type=jnp.float32)
    bb = bb_ref[0, 0]
    sc1 = jnp.sum(t * g, axis=1, keepdims=True) + bb
    sc2 = jnp.sum(ta * g, axis=1, keepdims=True) + bb
    sa1 = jnp.sum(ta * ga, axis=1, keepdims=True) + bb
    sa2 = jnp.sum(t * ga, axis=1, keepdims=True) + bb
    ret_ref[...] = jnp.concatenate([sc1, sc2], axis=1)
    reta_ref[...] = jnp.concatenate([sa1, sa2], axis=1)


@jax.jit
def kernel(feat, feat_a, adj, graph_neigh, W1, b1, W2, b2, Wb, bb):
    nblk = N // BM
    hidden = W1.shape[1]
    out_dim = W2.shape[1]

    dinv, y1s = pl.pallas_call(
        _prep_kernel,
        grid=(nblk,),
        in_specs=[
            pl.BlockSpec((BM, N), lambda i: (i, 0)),
            pl.BlockSpec((BM, feat.shape[1]), lambda i: (i, 0)),
            pl.BlockSpec((BM, feat.shape[1]), lambda i: (i, 0)),
            pl.BlockSpec(W1.shape, lambda i: (0, 0)),
        ],
        out_specs=[
            pl.BlockSpec((BM, 1), lambda i: (i, 0)),
            pl.BlockSpec((BM, 2 * hidden), lambda i: (i, 0)),
        ],
        out_shape=[
            jax.ShapeDtypeStruct((N, 1), jnp.float32),
            jax.ShapeDtypeStruct((N, 2 * hidden), jnp.float32),
        ],
    )(adj, feat, feat_a, W1)

    b1c = jnp.concatenate([b1, b1]).reshape(1, 2 * hidden)
    y2s = pl.pallas_call(
        _layer1_kernel,
        grid=(nblk,),
        in_specs=[
            pl.BlockSpec((BM, N), lambda i: (i, 0)),
            pl.BlockSpec((N, 2 * hidden), lambda i: (0, 0)),
            pl.BlockSpec((BM, 1), lambda i: (i, 0)),
            pl.BlockSpec((1, 2 * hidden), lambda i: (0, 0)),
            pl.BlockSpec(W2.shape, lambda i: (0, 0)),
        ],
        out_specs=pl.BlockSpec((BM, 2 * out_dim), lambda i: (i, 0)),
        out_shape=jax.ShapeDtypeStruct((N, 2 * out_dim), jnp.float32),
    )(adj, y1s, dinv, b1c, W2)

    b2c = jnp.concatenate([b2, b2]).reshape(1, 2 * out_dim)
    embcat = pl.pallas_call(
        _layer2_kernel,
        grid=(nblk,),
        in_specs=[
            pl.BlockSpec((BM, N), lambda i: (i, 0)),
            pl.BlockSpec((N, 2 * out_dim), lambda i: (0, 0)),
            pl.BlockSpec((BM, 1), lambda i: (i, 0)),
            pl.BlockSpec((1, 2 * out_dim), lambda i: (0, 0)),
        ],
        out_specs=pl.BlockSpec((BM, 2 * out_dim), lambda i: (i, 0)),
        out_shape=jax.ShapeDtypeStruct((N, 2 * out_dim), jnp.float32),
    )(adj, y2s, dinv, b2c)

    ret, ret_a = pl.pallas_call(
        _readout_kernel,
        grid=(nblk,),
        in_specs=[
            pl.BlockSpec((BM, N), lambda i: (i, 0)),
            pl.BlockSpec((N, 2 * out_dim), lambda i: (0, 0)),
            pl.BlockSpec((BM, 2 * out_dim), lambda i: (i, 0)),
            pl.BlockSpec(Wb.shape, lambda i: (0, 0)),
            pl.BlockSpec((1, 1), lambda i: (0, 0)),
        ],
        out_specs=[
            pl.BlockSpec((BM, 2), lambda i: (i, 0)),
            pl.BlockSpec((BM, 2), lambda i: (i, 0)),
        ],
        out_shape=[
            jax.ShapeDtypeStruct((N, 2), jnp.float32),
            jax.ShapeDtypeStruct((N, 2), jnp.float32),
        ],
    )(graph_neigh, embcat, embcat, Wb, bb.reshape(1, 1))

    emb = embcat[:, :out_dim]
    return (emb, ret, ret_a)
